# Initial kernel scaffold; baseline (speedup 1.0000x reference)
import jax
import jax.numpy as jnp
from jax.experimental import pallas as pl


def kernel(x, gain, bias, log_sigma_raw, log_mix, log_scale):
    raise NotImplementedError("write your pallas kernel here")


# speedup vs baseline: 11.6119x; 11.6119x over previous
import functools

import jax
import jax.numpy as jnp
from jax.experimental import pallas as pl
from jax.experimental.pallas import tpu as pltpu

_K_HIGH = 16
_K_LOW = 2


def _block_body(sig_ref, mix_ref, scl_ref, x_ref, gain_ref, bias_ref,
                out_ref, sim_ref, a_ref, *, bt: int, t: int, d: int):
    i = pl.program_id(1)
    xk = x_ref[0]
    q = x_ref[0, pl.ds(i * bt, bt), :]

    kn = xk / jnp.clip(jnp.sqrt(jnp.sum(xk * xk, axis=1, keepdims=True)),
                       1e-12, None)
    qn = q / jnp.clip(jnp.sqrt(jnp.sum(q * q, axis=1, keepdims=True)),
                      1e-12, None)

    sim = jax.lax.dot_general(qn, kn, (((1,), (1,)), ((), ())),
                              preferred_element_type=jnp.float32)

    iota_s = jax.lax.broadcasted_iota(jnp.int32, (bt, t), 1)
    t_glob = i * bt + jax.lax.broadcasted_iota(jnp.int32, (bt, t), 0)
    sim_ref[...] = jnp.where(iota_s < t_glob, sim, jnp.float32(-1e9))

    sigma = sig_ref[0, 0]
    surp = jnp.tanh(sigma * jnp.mean(jnp.abs(q), axis=1, keepdims=True))
    kt = jnp.clip(jnp.round(_K_LOW + (_K_HIGH - _K_LOW) * surp),
                  0.0, float(min(_K_HIGH, t - 1)))

    a_ref[...] = jnp.zeros((bt, t), jnp.float32)
    deg = jnp.zeros((bt, 1), jnp.float32)
    for j in range(_K_HIGH):
        s = sim_ref[...]
        cur = jnp.max(s, axis=1, keepdims=True)
        ismax = s == cur
        idxs = jnp.min(jnp.where(ismax, iota_s, t), axis=1, keepdims=True)
        onehot = iota_s == idxs
        sel = jnp.logical_and(kt > j, cur > -1e8)
        a_ref[...] += jnp.where(jnp.logical_and(onehot, sel), 1.0, 0.0)
        sim_ref[...] = jnp.where(onehot, jnp.float32(-2e9), s)
        deg = deg + jnp.where(sel, 1.0, 0.0)

    msg = jax.lax.dot_general(a_ref[...], xk, (((1,), (0,)), ((), ())),
                              preferred_element_type=jnp.float32)
    msg = msg / jnp.maximum(deg, 1.0)

    mix = mix_ref[0, 0]
    scale = scl_ref[0, 0]
    blended = mix * q + (1.0 - mix) * msg
    y = blended * gain_ref[0] + bias_ref[0]
    gelu = 0.5 * y * (1.0 + jax.lax.erf(y * jnp.float32(0.7071067811865476)))
    out_ref[0] = gelu * scale


@functools.partial(jax.jit, static_argnames=("interpret",))
def kernel(x, gain, bias, log_sigma_raw, log_mix, log_scale,
           interpret: bool = False):
    b, t, d = x.shape
    bt = 256
    ni = t // bt

    sigma = (jax.nn.softplus(log_sigma_raw) + 0.01).reshape(1, 1)
    mix = jax.nn.sigmoid(log_mix).reshape(1, 1)
    scale = (jax.nn.softplus(log_scale) + 0.01).reshape(1, 1)

    grid = (b, ni)
    out = pl.pallas_call(
        functools.partial(_block_body, bt=bt, t=t, d=d),
        grid=grid,
        in_specs=[
            pl.BlockSpec((1, 1), lambda bb, ii: (0, 0),
                         memory_space=pltpu.SMEM),
            pl.BlockSpec((1, 1), lambda bb, ii: (0, 0),
                         memory_space=pltpu.SMEM),
            pl.BlockSpec((1, 1), lambda bb, ii: (0, 0),
                         memory_space=pltpu.SMEM),
            pl.BlockSpec((1, t, d), lambda bb, ii: (bb, 0, 0)),
            pl.BlockSpec((1, d), lambda bb, ii: (0, 0)),
            pl.BlockSpec((1, d), lambda bb, ii: (0, 0)),
        ],
        out_specs=pl.BlockSpec((1, bt, d), lambda bb, ii: (bb, ii, 0)),
        out_shape=jax.ShapeDtypeStruct((b, t, d), jnp.float32),
        scratch_shapes=[
            pltpu.VMEM((bt, t), jnp.float32),
            pltpu.VMEM((bt, t), jnp.float32),
        ],
        interpret=interpret,
    )(sigma.astype(jnp.float32), mix.astype(jnp.float32),
      scale.astype(jnp.float32), x,
      gain.reshape(1, d), bias.reshape(1, d))
    return out


# speedup vs baseline: 15.2581x; 1.3140x over previous
import functools

import jax
import jax.numpy as jnp
from jax.experimental import pallas as pl
from jax.experimental.pallas import tpu as pltpu

_K_HIGH = 16
_K_LOW = 2


def _block_body(sig_ref, mix_ref, scl_ref, x_ref, gain_ref, bias_ref,
                out_ref, sim_ref, *, bt: int, t: int, d: int):
    i = pl.program_id(1)
    xk = x_ref[0]
    q = x_ref[0, pl.ds(i * bt, bt), :]

    kn = xk / jnp.clip(jnp.sqrt(jnp.sum(xk * xk, axis=1, keepdims=True)),
                       1e-12, None)
    qn = q / jnp.clip(jnp.sqrt(jnp.sum(q * q, axis=1, keepdims=True)),
                      1e-12, None)

    sim = jax.lax.dot_general(qn, kn, (((1,), (1,)), ((), ())),
                              preferred_element_type=jnp.float32)

    iota_s = jax.lax.broadcasted_iota(jnp.int32, (bt, t), 1)
    t_glob = i * bt + jax.lax.broadcasted_iota(jnp.int32, (bt, t), 0)
    past = iota_s < t_glob
    s0 = jnp.where(past, sim, jnp.float32(-1e9))
    sim_ref[...] = s0

    sigma = sig_ref[0, 0]
    surp = jnp.tanh(sigma * jnp.mean(jnp.abs(q), axis=1, keepdims=True))
    kt = jnp.clip(jnp.round(_K_LOW + (_K_HIGH - _K_LOW) * surp),
                  0.0, float(min(_K_HIGH, t - 1)))

    vals, idxs = [], []
    for j in range(_K_HIGH):
        s = sim_ref[...]
        cur = jnp.max(s, axis=1, keepdims=True)
        idx = jnp.argmax(s, axis=1).reshape(bt, 1)
        sim_ref[...] = jnp.where(iota_s == idx, jnp.float32(-2e9), s)
        vals.append(cur)
        idxs.append(idx)
    v16 = jnp.concatenate(vals, axis=1)
    i16 = jnp.concatenate(idxs, axis=1)

    jj = jax.lax.broadcasted_iota(jnp.int32, (bt, _K_HIGH), 1)
    kti = kt.astype(jnp.int32)
    sel = jnp.logical_and(jj < kti, v16 > -1e8)
    deg = jnp.maximum(jnp.sum(sel.astype(jnp.float32), axis=1,
                              keepdims=True), 1.0)
    isk = jj == (kti - 1)
    vstar = jnp.sum(jnp.where(isk, v16, 0.0), axis=1, keepdims=True)
    istar = jnp.max(jnp.where(isk, i16, -1), axis=1, keepdims=True)

    a = jnp.logical_and(
        jnp.logical_or(sim > vstar,
                       jnp.logical_and(sim == vstar, iota_s <= istar)),
        past).astype(jnp.float32)

    msg = jax.lax.dot_general(a, xk, (((1,), (0,)), ((), ())),
                              preferred_element_type=jnp.float32)
    msg = msg / deg

    mix = mix_ref[0, 0]
    scale = scl_ref[0, 0]
    blended = mix * q + (1.0 - mix) * msg
    y = blended * gain_ref[0] + bias_ref[0]
    gelu = 0.5 * y * (1.0 + jax.lax.erf(y * jnp.float32(0.7071067811865476)))
    out_ref[0] = gelu * scale


@functools.partial(jax.jit, static_argnames=("interpret",))
def kernel(x, gain, bias, log_sigma_raw, log_mix, log_scale,
           interpret: bool = False):
    b, t, d = x.shape
    bt = 256
    ni = t // bt

    sigma = (jax.nn.softplus(log_sigma_raw) + 0.01).reshape(1, 1)
    mix = jax.nn.sigmoid(log_mix).reshape(1, 1)
    scale = (jax.nn.softplus(log_scale) + 0.01).reshape(1, 1)

    grid = (b, ni)
    out = pl.pallas_call(
        functools.partial(_block_body, bt=bt, t=t, d=d),
        grid=grid,
        in_specs=[
            pl.BlockSpec((1, 1), lambda bb, ii: (0, 0),
                         memory_space=pltpu.SMEM),
            pl.BlockSpec((1, 1), lambda bb, ii: (0, 0),
                         memory_space=pltpu.SMEM),
            pl.BlockSpec((1, 1), lambda bb, ii: (0, 0),
                         memory_space=pltpu.SMEM),
            pl.BlockSpec((1, t, d), lambda bb, ii: (bb, 0, 0)),
            pl.BlockSpec((1, d), lambda bb, ii: (0, 0)),
            pl.BlockSpec((1, d), lambda bb, ii: (0, 0)),
        ],
        out_specs=pl.BlockSpec((1, bt, d), lambda bb, ii: (bb, ii, 0)),
        out_shape=jax.ShapeDtypeStruct((b, t, d), jnp.float32),
        scratch_shapes=[
            pltpu.VMEM((bt, t), jnp.float32),
        ],
        interpret=interpret,
    )(sigma.astype(jnp.float32), mix.astype(jnp.float32),
      scale.astype(jnp.float32), x,
      gain.reshape(1, d), bias.reshape(1, d))
    return out
